# initial kernel scaffold (unmeasured)
import functools

import jax
import jax.numpy as jnp
from jax import lax
from jax.experimental import pallas as pl
from jax.experimental.pallas import tpu as pltpu

N_DEV = 8
P = 160


def _body(off_ref, x_ref, cnt_ref, data_ref, cmat_ref,
          send_sems, recv_sems, csend_sems, crecv_sems, local_sem):
    my = lax.axis_index("i")

    barrier = pltpu.get_barrier_semaphore()
    for k in range(1, N_DEV):
        peer = lax.rem(my + k, N_DEV)
        pl.semaphore_signal(
            barrier, inc=1,
            device_id=(peer,), device_id_type=pl.DeviceIdType.MESH,
        )
    pl.semaphore_wait(barrier, N_DEV - 1)

    loc_data = pltpu.make_async_copy(
        x_ref.at[pl.ds(off_ref[my], P)], data_ref.at[my], local_sem)
    loc_data.start()
    loc_cnt = pltpu.make_async_copy(cnt_ref.at[0], cmat_ref.at[my], local_sem)

    sends = []
    for k in range(1, N_DEV):
        dst = lax.rem(my + k, N_DEV)
        rdma = pltpu.make_async_remote_copy(
            src_ref=x_ref.at[pl.ds(off_ref[dst], P)],
            dst_ref=data_ref.at[my],
            send_sem=send_sems.at[k - 1],
            recv_sem=recv_sems.at[k - 1],
            device_id=(dst,),
            device_id_type=pl.DeviceIdType.MESH,
        )
        rdma.start()
        crdma = pltpu.make_async_remote_copy(
            src_ref=cnt_ref.at[0],
            dst_ref=cmat_ref.at[my],
            send_sem=csend_sems.at[k - 1],
            recv_sem=crecv_sems.at[k - 1],
            device_id=(dst,),
            device_id_type=pl.DeviceIdType.MESH,
        )
        crdma.start()
        sends.append((rdma, crdma))

    loc_data.wait()
    loc_cnt.start()

    for k in range(1, N_DEV):
        src = lax.rem(my - k + N_DEV, N_DEV)
        pltpu.make_async_remote_copy(
            src_ref=x_ref.at[pl.ds(0, P)],
            dst_ref=data_ref.at[src],
            send_sem=send_sems.at[k - 1],
            recv_sem=recv_sems.at[k - 1],
            device_id=(src,),
            device_id_type=pl.DeviceIdType.MESH,
        ).wait_recv()
        pltpu.make_async_remote_copy(
            src_ref=cnt_ref.at[0],
            dst_ref=cmat_ref.at[src],
            send_sem=csend_sems.at[k - 1],
            recv_sem=crecv_sems.at[k - 1],
            device_id=(src,),
            device_id_type=pl.DeviceIdType.MESH,
        ).wait_recv()

    for rdma, crdma in sends:
        rdma.wait_send()
        crdma.wait_send()
    loc_cnt.wait()


def kernel(x, dest):
    m, n = x.shape

    perm = jnp.argsort(dest, stable=True)
    x_sorted = x[perm]
    counts = jnp.sum(
        dest[None, :] == jnp.arange(N_DEV, dtype=dest.dtype)[:, None],
        axis=1, dtype=jnp.int32)
    offsets = (jnp.cumsum(counts) - counts).astype(jnp.int32)
    x_pad = jnp.concatenate([x_sorted, jnp.zeros((P, n), x.dtype)], axis=0)
    counts_row = jnp.zeros((1, 128), jnp.int32).at[0, :N_DEV].set(counts)

    data, cmat = pl.pallas_call(
        _body,
        out_shape=(
            jax.ShapeDtypeStruct((N_DEV, P, n), x.dtype),
            jax.ShapeDtypeStruct((N_DEV, 128), jnp.int32),
        ),
        in_specs=[
            pl.BlockSpec(memory_space=pltpu.SMEM),
            pl.BlockSpec(memory_space=pltpu.VMEM),
            pl.BlockSpec(memory_space=pltpu.VMEM),
        ],
        out_specs=(
            pl.BlockSpec(memory_space=pltpu.VMEM),
            pl.BlockSpec(memory_space=pltpu.VMEM),
        ),
        scratch_shapes=[
            pltpu.SemaphoreType.DMA((N_DEV - 1,)),
            pltpu.SemaphoreType.DMA((N_DEV - 1,)),
            pltpu.SemaphoreType.DMA((N_DEV - 1,)),
            pltpu.SemaphoreType.DMA((N_DEV - 1,)),
            pltpu.SemaphoreType.DMA,
        ],
        compiler_params=pltpu.CompilerParams(collective_id=0),
    )(offsets, x_pad, counts_row)

    my = lax.axis_index("i")
    sizes = jnp.take(cmat[:, :N_DEV], my, axis=1)
    cum = jnp.cumsum(sizes)
    starts = cum - sizes
    r = jnp.arange(m, dtype=jnp.int32)
    s_idx = jnp.searchsorted(cum, r, side="right").astype(jnp.int32)
    j = r - starts[s_idx]
    return data[s_idx, j]


# baseline (device time: 125655 ns/iter reference)
import functools

import jax
import jax.numpy as jnp
from jax import lax
from jax.experimental import pallas as pl
from jax.experimental.pallas import tpu as pltpu

N_DEV = 8
P = 160


def _body(x_ref, cnt_ref, data_ref, cmat_ref,
          send_sems, recv_sems, csend_sems, crecv_sems, local_sem):
    my = lax.axis_index("i")

    barrier = pltpu.get_barrier_semaphore()
    for k in range(1, N_DEV):
        peer = lax.rem(my + k, N_DEV)
        pl.semaphore_signal(
            barrier, inc=1,
            device_id=(peer,), device_id_type=pl.DeviceIdType.MESH,
        )
    pl.semaphore_wait(barrier, N_DEV - 1)

    loc_data = pltpu.make_async_copy(
        x_ref.at[my], data_ref.at[my], local_sem)
    loc_data.start()
    loc_cnt = pltpu.make_async_copy(cnt_ref.at[0], cmat_ref.at[my], local_sem)

    sends = []
    for k in range(1, N_DEV):
        dst = lax.rem(my + k, N_DEV)
        rdma = pltpu.make_async_remote_copy(
            src_ref=x_ref.at[dst],
            dst_ref=data_ref.at[my],
            send_sem=send_sems.at[k - 1],
            recv_sem=recv_sems.at[k - 1],
            device_id=(dst,),
            device_id_type=pl.DeviceIdType.MESH,
        )
        rdma.start()
        crdma = pltpu.make_async_remote_copy(
            src_ref=cnt_ref.at[0],
            dst_ref=cmat_ref.at[my],
            send_sem=csend_sems.at[k - 1],
            recv_sem=crecv_sems.at[k - 1],
            device_id=(dst,),
            device_id_type=pl.DeviceIdType.MESH,
        )
        crdma.start()
        sends.append((rdma, crdma))

    loc_data.wait()
    loc_cnt.start()

    for k in range(1, N_DEV):
        src = lax.rem(my - k + N_DEV, N_DEV)
        pltpu.make_async_remote_copy(
            src_ref=x_ref.at[0],
            dst_ref=data_ref.at[src],
            send_sem=send_sems.at[k - 1],
            recv_sem=recv_sems.at[k - 1],
            device_id=(src,),
            device_id_type=pl.DeviceIdType.MESH,
        ).wait_recv()
        pltpu.make_async_remote_copy(
            src_ref=cnt_ref.at[0],
            dst_ref=cmat_ref.at[src],
            send_sem=csend_sems.at[k - 1],
            recv_sem=crecv_sems.at[k - 1],
            device_id=(src,),
            device_id_type=pl.DeviceIdType.MESH,
        ).wait_recv()

    for rdma, crdma in sends:
        rdma.wait_send()
        crdma.wait_send()
    loc_cnt.wait()


def kernel(x, dest):
    m, n = x.shape

    perm = jnp.argsort(dest, stable=True)
    x_sorted = x[perm]
    counts = jnp.sum(
        dest[None, :] == jnp.arange(N_DEV, dtype=dest.dtype)[:, None],
        axis=1, dtype=jnp.int32)
    offsets = (jnp.cumsum(counts) - counts).astype(jnp.int32)
    gidx = jnp.minimum(
        offsets[:, None] + jnp.arange(P, dtype=jnp.int32)[None, :], m - 1)
    x_blocks = x_sorted[gidx]
    counts_row = jnp.zeros((1, 128), jnp.int32).at[0, :N_DEV].set(counts)

    data, cmat = pl.pallas_call(
        _body,
        out_shape=(
            jax.ShapeDtypeStruct((N_DEV, P, n), x.dtype),
            jax.ShapeDtypeStruct((N_DEV, 128), jnp.int32),
        ),
        in_specs=[
            pl.BlockSpec(memory_space=pltpu.VMEM),
            pl.BlockSpec(memory_space=pltpu.VMEM),
        ],
        out_specs=(
            pl.BlockSpec(memory_space=pltpu.VMEM),
            pl.BlockSpec(memory_space=pltpu.VMEM),
        ),
        scratch_shapes=[
            pltpu.SemaphoreType.DMA((N_DEV - 1,)),
            pltpu.SemaphoreType.DMA((N_DEV - 1,)),
            pltpu.SemaphoreType.DMA((N_DEV - 1,)),
            pltpu.SemaphoreType.DMA((N_DEV - 1,)),
            pltpu.SemaphoreType.DMA,
        ],
        compiler_params=pltpu.CompilerParams(collective_id=0),
    )(x_blocks, counts_row)

    my = lax.axis_index("i")
    sizes = jnp.take(cmat[:, :N_DEV], my, axis=1)
    cum = jnp.cumsum(sizes)
    starts = cum - sizes
    r = jnp.arange(m, dtype=jnp.int32)
    s_idx = jnp.searchsorted(cum, r, side="right").astype(jnp.int32)
    j = r - starts[s_idx]
    return data[s_idx, j]


# device time: 37392 ns/iter; 3.3605x vs baseline; 3.3605x over previous
import jax
import jax.numpy as jnp
from jax import lax
from jax.experimental import pallas as pl
from jax.experimental.pallas import tpu as pltpu

N_DEV = 8
P = 160


def _body(x_ref, slot_ref, cnt_ref, out_ref,
          comm_ref, recv_ref, cmat_ref,
          send_sems, recv_sems, csend_sems, crecv_sems, local_sem):
    m, n = x_ref.shape
    my = lax.axis_index("i")

    barrier = pltpu.get_barrier_semaphore()
    for k in range(1, N_DEV):
        peer = lax.rem(my + k, N_DEV)
        pl.semaphore_signal(
            barrier, inc=1,
            device_id=(peer,), device_id_type=pl.DeviceIdType.MESH,
        )
    pl.semaphore_wait(barrier, N_DEV - 1)

    csends = []
    for k in range(1, N_DEV):
        dst = lax.rem(my + k, N_DEV)
        crdma = pltpu.make_async_remote_copy(
            src_ref=cnt_ref.at[0],
            dst_ref=cmat_ref.at[my],
            send_sem=csend_sems.at[k - 1],
            recv_sem=crecv_sems.at[k - 1],
            device_id=(dst,),
            device_id_type=pl.DeviceIdType.MESH,
        )
        crdma.start()
        csends.append(crdma)
    loc_cnt = pltpu.make_async_copy(cnt_ref.at[0], cmat_ref.at[my], local_sem)
    loc_cnt.start()

    x = x_ref[:, :]
    slot = slot_ref[:, :]
    j_iota = lax.broadcasted_iota(jnp.int32, (P, m), 0)
    sends = []
    for k in range(1, N_DEV + 1):
        dst = lax.rem(my + k, N_DEV)
        mask = (slot == dst * P + j_iota).astype(x.dtype)
        block = jax.lax.dot_general(
            mask, x, (((1,), (0,)), ((), ())),
            preferred_element_type=jnp.float32)
        comm_ref[k - 1] = block
        if k < N_DEV:
            rdma = pltpu.make_async_remote_copy(
                src_ref=comm_ref.at[k - 1],
                dst_ref=recv_ref.at[my],
                send_sem=send_sems.at[k - 1],
                recv_sem=recv_sems.at[k - 1],
                device_id=(dst,),
                device_id_type=pl.DeviceIdType.MESH,
            )
            rdma.start()
            sends.append(rdma)
        else:
            loc_data = pltpu.make_async_copy(
                comm_ref.at[k - 1], recv_ref.at[my], local_sem)
            loc_data.start()
            loc_data.wait()

    loc_cnt.wait()
    for k in range(1, N_DEV):
        src = lax.rem(my - k + N_DEV, N_DEV)
        pltpu.make_async_remote_copy(
            src_ref=cnt_ref.at[0],
            dst_ref=cmat_ref.at[src],
            send_sem=csend_sems.at[k - 1],
            recv_sem=crecv_sems.at[k - 1],
            device_id=(src,),
            device_id_type=pl.DeviceIdType.MESH,
        ).wait_recv()

    my_onehot = (lax.broadcasted_iota(jnp.int32, (1, N_DEV), 1) == my
                 ).astype(jnp.int32)
    sizes = []
    for s in range(N_DEV):
        sizes.append(jnp.sum(cmat_ref[s:s + 1, :N_DEV] * my_onehot))

    for k in range(1, N_DEV):
        src = lax.rem(my - k + N_DEV, N_DEV)
        pltpu.make_async_remote_copy(
            src_ref=comm_ref.at[0],
            dst_ref=recv_ref.at[src],
            send_sem=send_sems.at[k - 1],
            recv_sem=recv_sems.at[k - 1],
            device_id=(src,),
            device_id_type=pl.DeviceIdType.MESH,
        ).wait_recv()

    r_iota = lax.broadcasted_iota(jnp.int32, (m, P), 0)
    jj_iota = lax.broadcasted_iota(jnp.int32, (m, P), 1)
    start = jnp.int32(0)
    acc = jnp.zeros((m, n), jnp.float32)
    for s in range(N_DEV):
        mask = ((r_iota == start + jj_iota) & (jj_iota < sizes[s])
                ).astype(jnp.float32)
        acc = acc + jax.lax.dot_general(
            mask, recv_ref[s], (((1,), (0,)), ((), ())),
            preferred_element_type=jnp.float32)
        start = start + sizes[s]
    out_ref[:, :] = acc

    for rdma in sends:
        rdma.wait_send()
    for crdma in csends:
        crdma.wait_send()


def kernel(x, dest):
    m, n = x.shape

    D = (dest[:, None] == jnp.arange(N_DEV, dtype=dest.dtype)[None, :]
         ).astype(jnp.int32)
    cum_excl = jnp.cumsum(D, axis=0) - D
    rank = jnp.sum(D * cum_excl, axis=1)
    slot = (dest.astype(jnp.int32) * P + rank).reshape(1, m)
    counts = jnp.sum(D, axis=0, dtype=jnp.int32)
    counts_row = jnp.zeros((1, 128), jnp.int32).at[0, :N_DEV].set(counts)

    return pl.pallas_call(
        _body,
        out_shape=jax.ShapeDtypeStruct((m, n), x.dtype),
        in_specs=[
            pl.BlockSpec(memory_space=pltpu.VMEM),
            pl.BlockSpec(memory_space=pltpu.VMEM),
            pl.BlockSpec(memory_space=pltpu.VMEM),
        ],
        out_specs=pl.BlockSpec(memory_space=pltpu.VMEM),
        scratch_shapes=[
            pltpu.VMEM((N_DEV, P, n), jnp.float32),
            pltpu.VMEM((N_DEV, P, n), jnp.float32),
            pltpu.VMEM((N_DEV, 128), jnp.int32),
            pltpu.SemaphoreType.DMA((N_DEV - 1,)),
            pltpu.SemaphoreType.DMA((N_DEV - 1,)),
            pltpu.SemaphoreType.DMA((N_DEV - 1,)),
            pltpu.SemaphoreType.DMA((N_DEV - 1,)),
            pltpu.SemaphoreType.DMA,
        ],
        compiler_params=pltpu.CompilerParams(collective_id=0),
    )(x, slot, counts_row)


# device time: 35488 ns/iter; 3.5408x vs baseline; 1.0537x over previous
import jax
import jax.numpy as jnp
from jax import lax
from jax.experimental import pallas as pl
from jax.experimental.pallas import tpu as pltpu

N_DEV = 8
P = 160


def _body(x_ref, slot_ref, cnt_ref, out_ref,
          comm_ref, recv_ref, cmat_ref,
          send_sems, recv_sems, csend_sems, crecv_sems, local_sem):
    m, n = x_ref.shape
    my = lax.axis_index("i")

    barrier = pltpu.get_barrier_semaphore()
    for k in range(1, N_DEV):
        peer = lax.rem(my + k, N_DEV)
        pl.semaphore_signal(
            barrier, inc=1,
            device_id=(peer,), device_id_type=pl.DeviceIdType.MESH,
        )
    pl.semaphore_wait(barrier, N_DEV - 1)

    csends = []
    for k in range(1, N_DEV):
        dst = lax.rem(my + k, N_DEV)
        crdma = pltpu.make_async_remote_copy(
            src_ref=cnt_ref.at[0],
            dst_ref=cmat_ref.at[my],
            send_sem=csend_sems.at[k - 1],
            recv_sem=crecv_sems.at[k - 1],
            device_id=(dst,),
            device_id_type=pl.DeviceIdType.MESH,
        )
        crdma.start()
        csends.append(crdma)
    loc_cnt = pltpu.make_async_copy(cnt_ref.at[0], cmat_ref.at[my], local_sem)
    loc_cnt.start()

    x = x_ref[:, :]
    slot = slot_ref[:, :]
    j_iota = lax.broadcasted_iota(jnp.int32, (P, m), 0)
    sends = []
    for k in range(1, N_DEV + 1):
        dst = lax.rem(my + k, N_DEV)
        mask = (slot == dst * P + j_iota).astype(x.dtype)
        block = jax.lax.dot_general(
            mask, x, (((1,), (0,)), ((), ())),
            preferred_element_type=jnp.float32)
        comm_ref[k - 1] = block
        if k < N_DEV:
            rdma = pltpu.make_async_remote_copy(
                src_ref=comm_ref.at[k - 1],
                dst_ref=recv_ref.at[my],
                send_sem=send_sems.at[k - 1],
                recv_sem=recv_sems.at[k - 1],
                device_id=(dst,),
                device_id_type=pl.DeviceIdType.MESH,
            )
            rdma.start()
            sends.append(rdma)
        else:
            loc_data = pltpu.make_async_copy(
                comm_ref.at[k - 1], recv_ref.at[my], local_sem)
            loc_data.start()
            loc_data.wait()

    loc_cnt.wait()
    for k in range(1, N_DEV):
        src = lax.rem(my - k + N_DEV, N_DEV)
        pltpu.make_async_remote_copy(
            src_ref=cnt_ref.at[0],
            dst_ref=cmat_ref.at[src],
            send_sem=csend_sems.at[k - 1],
            recv_sem=crecv_sems.at[k - 1],
            device_id=(src,),
            device_id_type=pl.DeviceIdType.MESH,
        ).wait_recv()

    my_onehot = (lax.broadcasted_iota(jnp.int32, (N_DEV, N_DEV), 1) == my
                 ).astype(jnp.int32)
    col = cmat_ref[:, :N_DEV] * my_onehot
    s_iota = lax.broadcasted_iota(jnp.int32, (N_DEV, N_DEV), 0)

    r_iota = lax.broadcasted_iota(jnp.int32, (m, P), 0)
    jj_iota = lax.broadcasted_iota(jnp.int32, (m, P), 1)
    acc = jnp.zeros((m, n), jnp.float32)
    for k in range(N_DEV):
        src = lax.rem(my - k + N_DEV, N_DEV)
        if k > 0:
            pltpu.make_async_remote_copy(
                src_ref=comm_ref.at[0],
                dst_ref=recv_ref.at[src],
                send_sem=send_sems.at[k - 1],
                recv_sem=recv_sems.at[k - 1],
                device_id=(src,),
                device_id_type=pl.DeviceIdType.MESH,
            ).wait_recv()
        size_s = jnp.sum(col * (s_iota == src))
        start_s = jnp.sum(col * (s_iota < src))
        mask = ((r_iota == start_s + jj_iota) & (jj_iota < size_s)
                ).astype(jnp.float32)
        acc = acc + jax.lax.dot_general(
            mask, recv_ref[src], (((1,), (0,)), ((), ())),
            preferred_element_type=jnp.float32)
    out_ref[:, :] = acc

    for rdma in sends:
        rdma.wait_send()
    for crdma in csends:
        crdma.wait_send()


def kernel(x, dest):
    m, n = x.shape

    D = (dest[:, None] == jnp.arange(N_DEV, dtype=dest.dtype)[None, :]
         ).astype(jnp.int32)
    cum_excl = jnp.cumsum(D, axis=0) - D
    rank = jnp.sum(D * cum_excl, axis=1)
    slot = (dest.astype(jnp.int32) * P + rank).reshape(1, m)
    counts = jnp.sum(D, axis=0, dtype=jnp.int32)
    counts_row = jnp.zeros((1, 128), jnp.int32).at[0, :N_DEV].set(counts)

    return pl.pallas_call(
        _body,
        out_shape=jax.ShapeDtypeStruct((m, n), x.dtype),
        in_specs=[
            pl.BlockSpec(memory_space=pltpu.VMEM),
            pl.BlockSpec(memory_space=pltpu.VMEM),
            pl.BlockSpec(memory_space=pltpu.VMEM),
        ],
        out_specs=pl.BlockSpec(memory_space=pltpu.VMEM),
        scratch_shapes=[
            pltpu.VMEM((N_DEV, P, n), jnp.float32),
            pltpu.VMEM((N_DEV, P, n), jnp.float32),
            pltpu.VMEM((N_DEV, 128), jnp.int32),
            pltpu.SemaphoreType.DMA((N_DEV - 1,)),
            pltpu.SemaphoreType.DMA((N_DEV - 1,)),
            pltpu.SemaphoreType.DMA((N_DEV - 1,)),
            pltpu.SemaphoreType.DMA((N_DEV - 1,)),
            pltpu.SemaphoreType.DMA,
        ],
        compiler_params=pltpu.CompilerParams(collective_id=0),
    )(x, slot, counts_row)
